# baseline (speedup 1.0000x reference)
import functools

import jax
import jax.numpy as jnp
from jax import lax
from jax.experimental import pallas as pl
from jax.experimental.pallas import tpu as pltpu
from jax.experimental.pallas import tpu_sc as plsc

L = 16
CAPF = 256
TAU = 0.05


def _sc_topk_gather(dists, featsflat, maskw, B, N, D, K):
    info = plsc.get_sparse_core_info()
    NC = info.num_cores
    NW = NC * info.num_subcores
    RPW = B // NW
    NB = N // L
    KB = K // L
    NW4 = N // 4
    CB = N + L
    UA = 4

    mesh = plsc.VectorSubcoreMesh(core_axis_name="c", subcore_axis_name="s")

    @functools.partial(
        pl.kernel,
        mesh=mesh,
        out_type=[
            jax.ShapeDtypeStruct((B, K, D), jnp.float32),
            jax.ShapeDtypeStruct((B, K), jnp.int32),
        ],
        scratch_types=[
            pltpu.VMEM((N,), jnp.float32),
            pltpu.VMEM((N,), jnp.float32),
            pltpu.VMEM((N,), jnp.int8),
            pltpu.VMEM((N,), jnp.int8),
            pltpu.VMEM((NW4,), jnp.int32),
            pltpu.VMEM((K,), jnp.float32),
            pltpu.VMEM((L,), jnp.float32),
            pltpu.VMEM((CB,), jnp.float32),
            pltpu.VMEM((CB,), jnp.int32),
            pltpu.VMEM((L,), jnp.int32),
            pltpu.SMEM((1,), jnp.int32),
            pltpu.VMEM((K,), jnp.int32),
            pltpu.VMEM((K,), jnp.int32),
            pltpu.VMEM((K,), jnp.int32),
            pltpu.VMEM((K,), jnp.int32),
            pltpu.VMEM((K,), jnp.int32),
            pltpu.VMEM((K, D), jnp.float32),
            pltpu.VMEM((K, D), jnp.float32),
            pltpu.SemaphoreType.DMA,
            pltpu.SemaphoreType.DMA,
            pltpu.SemaphoreType.DMA,
            pltpu.SemaphoreType.DMA,
            pltpu.SemaphoreType.DMA,
            pltpu.SemaphoreType.DMA,
        ],
        compiler_params=pltpu.CompilerParams(needs_layout_passes=False),
    )
    def sc_fn(dists_hbm, feats_hbm, mask_hbm, outf_hbm, outm_hbm,
              drow0, drow1, mrow0, mrow1, mwords, best, tref, cbuf, ibuf,
              cw2ref, flag, idxl,
              idxg0, idxg1, mb0, mb1, rows0, rows1,
              semd0, semd1, semg0, semg1, semo0, semo1):
        cid = lax.axis_index("c")
        sid = lax.axis_index("s")
        wid = sid * NC + cid

        drow = [drow0, drow1]
        mrow = [mrow0, mrow1]
        idxg = [idxg0, idxg1]
        mb = [mb0, mb1]
        rows = [rows0, rows1]
        semd = [semd0, semd1]
        semg = [semg0, semg1]
        semo = [semo0, semo1]
        cp_in = [None, None]
        cp_g = [None, None]
        cp_o = [None, None]

        ii = lax.iota(jnp.int32, L)
        zero_i = jnp.zeros((L,), jnp.int32)
        one_i = jnp.full((L,), 1, jnp.int32)
        inf = jnp.full((L,), jnp.inf, jnp.float32)
        tauv = jnp.full((L,), TAU, jnp.float32)

        def fire_in(r, b):
            row = wid * RPW + r
            c1 = pltpu.async_copy(dists_hbm.at[row], drow[b], semd[b])
            c2 = pltpu.async_copy(
                mask_hbm.at[pl.ds(pl.multiple_of(row * N, N), N)],
                mrow[b], semd[b])
            cp_in[b] = (c1, c2)

        def merge(d):
            ns = lax.rev(lax.sort(d), (0,))
            b0 = best[pl.ds(0 * L, L)]
            b1 = best[pl.ds(1 * L, L)]
            b2 = best[pl.ds(2 * L, L)]
            l0 = jnp.minimum(b0, b2)
            h0 = jnp.maximum(b0, b2)
            l1 = jnp.minimum(b1, ns)
            h1 = jnp.maximum(b1, ns)
            a0 = jnp.minimum(l0, l1)
            a1 = jnp.maximum(l0, l1)
            a2 = jnp.minimum(h0, h1)
            nb2 = lax.sort(a2)
            best[pl.ds(0 * L, L)] = lax.sort(a0)
            best[pl.ds(1 * L, L)] = lax.sort(a1)
            best[pl.ds(2 * L, L)] = nb2
            tref[...] = jnp.full((L,), nb2[15], jnp.float32)

        def compute(r, b):
            row = wid * RPW + r
            db = drow[b]
            for j in range(KB):
                best[pl.ds(j * L, L)] = inf
            tref[...] = inf
            for j in range(CAPF // L + 1):
                cbuf[pl.ds(j * L, L)] = inf
            flag[0] = jnp.int32(0)

            G = 4

            def sweep1(g, carry):
                cwm1, gl = carry
                ds, les, cums_l = [], [], []
                for j in range(G):
                    d = db[pl.ds((g * G + j) * L, L)]
                    le = d <= tauv
                    lei = jnp.where(le, one_i, zero_i)
                    ds.append(d)
                    les.append(le)
                    cums_l.append(lax.cumsum(lei))
                offs = [cwm1]
                for j in range(G):
                    offs.append(
                        offs[j] + jnp.full((L,), cums_l[j][15], jnp.int32))
                for j in range(G):
                    pos = offs[j] + cums_l[j]
                    plsc.store_scatter(cbuf, [pos], ds[j], mask=les[j])
                    plsc.store_scatter(ibuf, [pos], gl + j * L, mask=les[j])
                return offs[G], gl + G * L
            cwm1, _ = lax.fori_loop(
                0, NB // G, sweep1, (zero_i - 1, ii))

            c_tau = cwm1[15] + 1
            ok1 = (c_tau >= K) & (c_tau <= CAPF)

            @pl.when(ok1)
            def _():
                ncb = (c_tau + (L - 1)) // L

                def ca(i, _):
                    merge(cbuf[pl.ds(i * L, L)])
                    return None
                lax.fori_loop(0, ncb, ca, None)

                tv = tref[...]
                cw2ref[...] = zero_i - 1

                def cb(i, _):
                    c = cbuf[pl.ds(i * L, L)]
                    le = c <= tv
                    lei = jnp.where(le, one_i, zero_i)
                    cums = lax.cumsum(lei)
                    cwm1c = cw2ref[...]
                    pos = cwm1c + cums
                    sel = le & (pos < K)
                    iv = ibuf[pl.ds(i * L, L)]
                    plsc.store_scatter(idxl, [pos], iv, mask=sel)
                    plsc.store_scatter(idxg[b], [pos], iv + row * N, mask=sel)
                    cw2ref[...] = cwm1c + jnp.full((L,), cums[15], jnp.int32)
                    return None
                lax.fori_loop(0, ncb, cb, None)
                flag[0] = (cw2ref[...][15] + 1 == K).astype(jnp.int32)

            @pl.when(flag[0] == 0)
            def _():
                for j in range(KB):
                    best[pl.ds(j * L, L)] = inf
                tref[...] = inf

                def pass_a(i, _):
                    tv = tref[...]
                    ds = [db[pl.ds((i * UA + j) * L, L)] for j in range(UA)]
                    ms = [d < tv for d in ds]
                    anyhit = (ms[0] | ms[1]) | (ms[2] | ms[3])
                    cnt = plsc.all_reduce_population_count(anyhit)

                    @pl.when(cnt[0] > 0)
                    def _():
                        for j in range(UA):
                            cj = plsc.all_reduce_population_count(ms[j])

                            @pl.when(cj[0] > 0)
                            def _(j=j):
                                merge(ds[j])

                    return None
                lax.fori_loop(0, NB // UA, pass_a, None)

                tv = tref[...]

                def b1(i, acc):
                    d = db[pl.ds(i * L, L)]
                    return acc + (d < tv).astype(jnp.int32)
                c_less = jnp.sum(
                    lax.fori_loop(0, NB, b1, zero_i))
                m = K - c_less

                def b2(i, carry):
                    cw, ct = carry
                    d = db[pl.ds(i * L, L)]
                    lt = d < tv
                    eq = d == tv
                    eqi = eq.astype(jnp.int32)
                    ranks = ct + lax.cumsum(eqi) - eqi
                    sel = lt | (eq & (ranks < m))
                    seli = sel.astype(jnp.int32)
                    pos = cw + lax.cumsum(seli) - seli
                    gl = i * L + ii
                    plsc.store_scatter(idxl, [pos], gl, mask=sel)
                    plsc.store_scatter(idxg[b], [pos], gl + row * N, mask=sel)
                    return cw + jnp.sum(seli), ct + jnp.sum(eqi)
                lax.fori_loop(0, NB, b2, (jnp.int32(0), jnp.int32(0)))

            def packw(i, _):
                bs = mrow[b][pl.ds(i * 64, 64)]
                mwords[pl.ds(i * L, L)] = plsc.bitcast(bs, jnp.int32)
                return None
            lax.fori_loop(0, NW4 // L, packw, None)
            for j in range(KB):
                iv = idxl[pl.ds(j * L, L)]
                w = plsc.load_gather(mwords, [jnp.right_shift(iv, 2)])
                sh = jnp.left_shift(iv & 3, 3)
                mb[b][pl.ds(j * L, L)] = jnp.right_shift(w, sh) & 1

        def fire_gather(b):
            cp_g[b] = pltpu.async_copy(feats_hbm.at[idxg[b]], rows[b], semg[b])

        def fire_out(r, b):
            row = wid * RPW + r
            c1 = pltpu.async_copy(rows[b], outf_hbm.at[row], semo[b])
            c2 = pltpu.async_copy(mb[b], outm_hbm.at[row], semo[b])
            cp_o[b] = (c1, c2)

        fire_in(0, 0)
        for r in range(RPW):
            b = r % 2
            if r + 1 < RPW:
                fire_in(r + 1, 1 - b)
            for c in cp_in[b]:
                c.wait()
            if r >= 2:
                for c in cp_o[b]:
                    c.wait()
            compute(r, b)
            fire_gather(b)
            if r >= 1:
                cp_g[1 - b].wait()
                fire_out(r - 1, 1 - b)
        lb = (RPW - 1) % 2
        cp_g[lb].wait()
        fire_out(RPW - 1, lb)
        for c in cp_o[1 - lb]:
            c.wait()
        for c in cp_o[lb]:
            c.wait()

    return sc_fn(dists, featsflat, maskw)


def kernel(dists, feats, coord_mask):
    B, N = dists.shape
    D = feats.shape[2]
    K = min(48, N)
    featsflat = feats.reshape(B * N, D)
    maskr = coord_mask.view(jnp.int8).reshape(B * N)
    outf, outm = _sc_topk_gather(dists, featsflat, maskr, B, N, D, K)
    return outf, outm != 0


# speedup vs baseline: 1.0562x; 1.0562x over previous
import functools

import jax
import jax.numpy as jnp
from jax import lax
from jax.experimental import pallas as pl
from jax.experimental.pallas import tpu as pltpu
from jax.experimental.pallas import tpu_sc as plsc

L = 16
CAPF = 256
TAU = 0.05


def _sc_topk_gather(dists, featsflat, maskw, B, N, D, K):
    info = plsc.get_sparse_core_info()
    NC = info.num_cores
    NW = NC * info.num_subcores
    RPW = B // NW
    NB = N // L
    KB = K // L
    NW4 = N // 4
    CB = N + L
    UA = 4

    mesh = plsc.VectorSubcoreMesh(core_axis_name="c", subcore_axis_name="s")

    @functools.partial(
        pl.kernel,
        mesh=mesh,
        out_type=[
            jax.ShapeDtypeStruct((B, K, D), jnp.float32),
            jax.ShapeDtypeStruct((B, K), jnp.int32),
        ],
        scratch_types=[
            pltpu.VMEM((N,), jnp.float32),
            pltpu.VMEM((N,), jnp.float32),
            pltpu.VMEM((N,), jnp.int8),
            pltpu.VMEM((N,), jnp.int8),
            pltpu.VMEM((NW4,), jnp.int32),
            pltpu.VMEM((K,), jnp.float32),
            pltpu.VMEM((L,), jnp.float32),
            pltpu.VMEM((CB,), jnp.float32),
            pltpu.VMEM((CB,), jnp.int32),
            pltpu.VMEM((L,), jnp.int32),
            pltpu.SMEM((1,), jnp.int32),
            pltpu.VMEM((K,), jnp.int32),
            pltpu.VMEM((K,), jnp.int32),
            pltpu.VMEM((K,), jnp.int32),
            pltpu.VMEM((K,), jnp.int32),
            pltpu.VMEM((K,), jnp.int32),
            pltpu.VMEM((K, D), jnp.float32),
            pltpu.VMEM((K, D), jnp.float32),
            pltpu.SemaphoreType.DMA,
            pltpu.SemaphoreType.DMA,
            pltpu.SemaphoreType.DMA,
            pltpu.SemaphoreType.DMA,
            pltpu.SemaphoreType.DMA,
            pltpu.SemaphoreType.DMA,
        ],
        compiler_params=pltpu.CompilerParams(needs_layout_passes=False),
    )
    def sc_fn(dists_hbm, feats_hbm, mask_hbm, outf_hbm, outm_hbm,
              drow0, drow1, mrow0, mrow1, mwords, best, tref, cbuf, ibuf,
              cw2ref, flag, idxl,
              idxg0, idxg1, mb0, mb1, rows0, rows1,
              semd0, semd1, semg0, semg1, semo0, semo1):
        cid = lax.axis_index("c")
        sid = lax.axis_index("s")
        wid = sid * NC + cid

        drow = [drow0, drow1]
        mrow = [mrow0, mrow1]
        idxg = [idxg0, idxg1]
        mb = [mb0, mb1]
        rows = [rows0, rows1]
        semd = [semd0, semd1]
        semg = [semg0, semg1]
        semo = [semo0, semo1]
        cp_in = [None, None]
        cp_g = [None, None]
        cp_o = [None, None]

        ii = lax.iota(jnp.int32, L)
        zero_i = jnp.zeros((L,), jnp.int32)
        one_i = jnp.full((L,), 1, jnp.int32)
        inf = jnp.full((L,), jnp.inf, jnp.float32)
        tauv = jnp.full((L,), TAU, jnp.float32)

        def fire_in(r, b):
            row = wid * RPW + r
            c1 = pltpu.async_copy(dists_hbm.at[row], drow[b], semd[b])
            c2 = pltpu.async_copy(
                mask_hbm.at[pl.ds(pl.multiple_of(row * N, N), N)],
                mrow[b], semd[b])
            cp_in[b] = (c1, c2)

        def merge(d):
            ns = lax.rev(lax.sort(d), (0,))
            b0 = best[pl.ds(0 * L, L)]
            b1 = best[pl.ds(1 * L, L)]
            b2 = best[pl.ds(2 * L, L)]
            l0 = jnp.minimum(b0, b2)
            h0 = jnp.maximum(b0, b2)
            l1 = jnp.minimum(b1, ns)
            h1 = jnp.maximum(b1, ns)
            a0 = jnp.minimum(l0, l1)
            a1 = jnp.maximum(l0, l1)
            a2 = jnp.minimum(h0, h1)
            nb2 = lax.sort(a2)
            best[pl.ds(0 * L, L)] = lax.sort(a0)
            best[pl.ds(1 * L, L)] = lax.sort(a1)
            best[pl.ds(2 * L, L)] = nb2
            tref[...] = jnp.full((L,), nb2[15], jnp.float32)

        def compute(r, b):
            row = wid * RPW + r
            db = drow[b]
            for j in range(KB):
                best[pl.ds(j * L, L)] = inf
            tref[...] = inf
            for j in range(CAPF // L + 1):
                cbuf[pl.ds(j * L, L)] = inf
            flag[0] = jnp.int32(0)

            G = 4

            def sweep1(g, carry):
                cwm1, gl = carry
                ds, les, cums_l = [], [], []
                for j in range(G):
                    d = db[pl.ds((g * G + j) * L, L)]
                    le = d <= tauv
                    lei = jnp.where(le, one_i, zero_i)
                    ds.append(d)
                    les.append(le)
                    cums_l.append(lax.cumsum(lei))
                offs = [cwm1]
                for j in range(G):
                    offs.append(
                        offs[j] + jnp.full((L,), cums_l[j][15], jnp.int32))
                for j in range(G):
                    pos = offs[j] + cums_l[j]
                    plsc.store_scatter(cbuf, [pos], ds[j], mask=les[j])
                    plsc.store_scatter(ibuf, [pos], gl + j * L, mask=les[j])
                return offs[G], gl + G * L
            cwm1, _ = lax.fori_loop(
                0, NB // G, sweep1, (zero_i - 1, ii))

            c_tau = cwm1[15] + 1
            ok1 = (c_tau >= K) & (c_tau <= CAPF)

            @pl.when(ok1)
            def _():
                ncb = (c_tau + (L - 1)) // L

                def ca(i, _):
                    merge(cbuf[pl.ds(i * L, L)])
                    return None
                lax.fori_loop(0, ncb, ca, None)

                tv = tref[...]
                cw2ref[...] = zero_i - 1

                def cb(i, _):
                    c = cbuf[pl.ds(i * L, L)]
                    le = c <= tv
                    lei = jnp.where(le, one_i, zero_i)
                    cums = lax.cumsum(lei)
                    cwm1c = cw2ref[...]
                    pos = cwm1c + cums
                    sel = le & (pos < K)
                    iv = ibuf[pl.ds(i * L, L)]
                    plsc.store_scatter(idxl, [pos], iv, mask=sel)
                    plsc.store_scatter(idxg[b], [pos], iv + row * N, mask=sel)
                    cw2ref[...] = cwm1c + jnp.full((L,), cums[15], jnp.int32)
                    return None
                lax.fori_loop(0, ncb, cb, None)
                flag[0] = (cw2ref[...][15] + 1 == K).astype(jnp.int32)

            @pl.when(flag[0] == 0)
            def _():
                for j in range(KB):
                    best[pl.ds(j * L, L)] = inf
                tref[...] = inf

                def pass_a(i, _):
                    tv = tref[...]
                    d = db[pl.ds(i * L, L)]
                    cnt = plsc.all_reduce_population_count(d < tv)

                    @pl.when(cnt[0] > 0)
                    def _():
                        merge(d)

                    return None
                lax.fori_loop(0, NB, pass_a, None)

                tv = tref[...]

                def b1(i, acc):
                    d = db[pl.ds(i * L, L)]
                    return acc + (d < tv).astype(jnp.int32)
                c_less = jnp.sum(
                    lax.fori_loop(0, NB, b1, zero_i))
                m = K - c_less

                def b2(i, carry):
                    cw, ct = carry
                    d = db[pl.ds(i * L, L)]
                    lt = d < tv
                    eq = d == tv
                    eqi = eq.astype(jnp.int32)
                    ranks = ct + lax.cumsum(eqi) - eqi
                    sel = lt | (eq & (ranks < m))
                    seli = sel.astype(jnp.int32)
                    pos = cw + lax.cumsum(seli) - seli
                    gl = i * L + ii
                    plsc.store_scatter(idxl, [pos], gl, mask=sel)
                    plsc.store_scatter(idxg[b], [pos], gl + row * N, mask=sel)
                    return cw + jnp.sum(seli), ct + jnp.sum(eqi)
                lax.fori_loop(0, NB, b2, (jnp.int32(0), jnp.int32(0)))

            def packw(i, _):
                bs = mrow[b][pl.ds(i * 64, 64)]
                mwords[pl.ds(i * L, L)] = plsc.bitcast(bs, jnp.int32)
                return None
            lax.fori_loop(0, NW4 // L, packw, None)
            for j in range(KB):
                iv = idxl[pl.ds(j * L, L)]
                w = plsc.load_gather(mwords, [jnp.right_shift(iv, 2)])
                sh = jnp.left_shift(iv & 3, 3)
                mb[b][pl.ds(j * L, L)] = jnp.right_shift(w, sh) & 1

        def fire_gather(b):
            cp_g[b] = pltpu.async_copy(feats_hbm.at[idxg[b]], rows[b], semg[b])

        def fire_out(r, b):
            row = wid * RPW + r
            c1 = pltpu.async_copy(rows[b], outf_hbm.at[row], semo[b])
            c2 = pltpu.async_copy(mb[b], outm_hbm.at[row], semo[b])
            cp_o[b] = (c1, c2)

        def wait_in(b):
            pltpu.make_async_copy(dists_hbm.at[0], drow[b], semd[b]).wait()
            pltpu.make_async_copy(
                mask_hbm.at[pl.ds(0, N)], mrow[b], semd[b]).wait()

        def wait_out(b):
            pltpu.make_async_copy(rows[b], outf_hbm.at[0], semo[b]).wait()
            pltpu.make_async_copy(mb[b], outm_hbm.at[0], semo[b]).wait()

        NIT = RPW // 2
        fire_in(0, 0)

        def row_pair(it, _):
            r0 = it * 2
            fire_in(r0 + 1, 1)
            wait_in(0)

            @pl.when(it > 0)
            def _():
                wait_out(0)

            compute(r0, 0)
            fire_gather(0)

            @pl.when(it < NIT - 1)
            def _():
                fire_in(r0 + 2, 0)

            wait_in(1)

            @pl.when(it > 0)
            def _():
                wait_out(1)

            compute(r0 + 1, 1)
            fire_gather(1)
            cp_g[0].wait()
            fire_out(r0, 0)
            cp_g[1].wait()
            fire_out(r0 + 1, 1)
            return None
        lax.fori_loop(0, NIT, row_pair, None)
        wait_out(0)
        wait_out(1)

    return sc_fn(dists, featsflat, maskw)


def kernel(dists, feats, coord_mask):
    B, N = dists.shape
    D = feats.shape[2]
    K = min(48, N)
    featsflat = feats.reshape(B * N, D)
    maskr = coord_mask.view(jnp.int8).reshape(B * N)
    outf, outm = _sc_topk_gather(dists, featsflat, maskr, B, N, D, K)
    return outf, outm != 0


# speedup vs baseline: 1.0583x; 1.0020x over previous
import functools

import jax
import jax.numpy as jnp
from jax import lax
from jax.experimental import pallas as pl
from jax.experimental.pallas import tpu as pltpu
from jax.experimental.pallas import tpu_sc as plsc

L = 16
CAPF = 256
TAU = 0.05


def _sc_topk_gather(dists, featsflat, maskw, B, N, D, K):
    info = plsc.get_sparse_core_info()
    NC = info.num_cores
    NW = NC * info.num_subcores
    RPW = B // NW
    NB = N // L
    KB = K // L
    NW4 = N // 4
    CB = N + L
    UA = 4

    mesh = plsc.VectorSubcoreMesh(core_axis_name="c", subcore_axis_name="s")

    @functools.partial(
        pl.kernel,
        mesh=mesh,
        out_type=[
            jax.ShapeDtypeStruct((B, K, D), jnp.float32),
            jax.ShapeDtypeStruct((B, K), jnp.int32),
        ],
        scratch_types=[
            pltpu.VMEM((N,), jnp.float32),
            pltpu.VMEM((N,), jnp.float32),
            pltpu.VMEM((N,), jnp.int8),
            pltpu.VMEM((N,), jnp.int8),
            pltpu.VMEM((NW4,), jnp.int32),
            pltpu.VMEM((K,), jnp.float32),
            pltpu.VMEM((L,), jnp.float32),
            pltpu.VMEM((CB,), jnp.float32),
            pltpu.VMEM((CB,), jnp.int32),
            pltpu.VMEM((L,), jnp.int32),
            pltpu.SMEM((1,), jnp.int32),
            pltpu.VMEM((K,), jnp.int32),
            pltpu.VMEM((K,), jnp.int32),
            pltpu.VMEM((K,), jnp.int32),
            pltpu.VMEM((K,), jnp.int32),
            pltpu.VMEM((K,), jnp.int32),
            pltpu.VMEM((K, D), jnp.float32),
            pltpu.VMEM((K, D), jnp.float32),
            pltpu.SemaphoreType.DMA,
            pltpu.SemaphoreType.DMA,
            pltpu.SemaphoreType.DMA,
            pltpu.SemaphoreType.DMA,
            pltpu.SemaphoreType.DMA,
            pltpu.SemaphoreType.DMA,
        ],
        compiler_params=pltpu.CompilerParams(needs_layout_passes=False),
    )
    def sc_fn(dists_hbm, feats_hbm, mask_hbm, outf_hbm, outm_hbm,
              drow0, drow1, mrow0, mrow1, mwords, best, tref, cbuf, ibuf,
              cw2ref, flag, idxl,
              idxg0, idxg1, mb0, mb1, rows0, rows1,
              semd0, semd1, semg0, semg1, semo0, semo1):
        cid = lax.axis_index("c")
        sid = lax.axis_index("s")
        wid = sid * NC + cid

        drow = [drow0, drow1]
        mrow = [mrow0, mrow1]
        idxg = [idxg0, idxg1]
        mb = [mb0, mb1]
        rows = [rows0, rows1]
        semd = [semd0, semd1]
        semg = [semg0, semg1]
        semo = [semo0, semo1]
        cp_in = [None, None]
        cp_g = [None, None]
        cp_o = [None, None]

        ii = lax.iota(jnp.int32, L)
        zero_i = jnp.zeros((L,), jnp.int32)
        one_i = jnp.full((L,), 1, jnp.int32)
        inf = jnp.full((L,), jnp.inf, jnp.float32)
        tauv = jnp.full((L,), TAU, jnp.float32)

        def fire_in(r, b):
            row = wid * RPW + r
            c1 = pltpu.async_copy(dists_hbm.at[row], drow[b], semd[b])
            c2 = pltpu.async_copy(
                mask_hbm.at[pl.ds(pl.multiple_of(row * N, N), N)],
                mrow[b], semd[b])
            cp_in[b] = (c1, c2)

        def merge(d):
            ns = lax.rev(lax.sort(d), (0,))
            b0 = best[pl.ds(0 * L, L)]
            b1 = best[pl.ds(1 * L, L)]
            b2 = best[pl.ds(2 * L, L)]
            l0 = jnp.minimum(b0, b2)
            h0 = jnp.maximum(b0, b2)
            l1 = jnp.minimum(b1, ns)
            h1 = jnp.maximum(b1, ns)
            a0 = jnp.minimum(l0, l1)
            a1 = jnp.maximum(l0, l1)
            a2 = jnp.minimum(h0, h1)
            nb2 = lax.sort(a2)
            best[pl.ds(0 * L, L)] = lax.sort(a0)
            best[pl.ds(1 * L, L)] = lax.sort(a1)
            best[pl.ds(2 * L, L)] = nb2
            tref[...] = jnp.full((L,), nb2[15], jnp.float32)

        def compute(r, b):
            row = wid * RPW + r
            db = drow[b]
            for j in range(KB):
                best[pl.ds(j * L, L)] = inf
            tref[...] = inf
            for j in range(CAPF // L + 1):
                cbuf[pl.ds(j * L, L)] = inf
            flag[0] = jnp.int32(0)

            G = 4

            def sweep1(g, carry):
                cwm1, gl = carry
                ds, les, cums_l = [], [], []
                for j in range(G):
                    d = db[pl.ds((g * G + j) * L, L)]
                    le = d <= tauv
                    lei = jnp.where(le, one_i, zero_i)
                    ds.append(d)
                    les.append(le)
                    cums_l.append(lax.cumsum(lei))
                offs = [cwm1]
                for j in range(G):
                    offs.append(
                        offs[j] + jnp.full((L,), cums_l[j][15], jnp.int32))
                for j in range(G):
                    pos = offs[j] + cums_l[j]
                    plsc.store_scatter(cbuf, [pos], ds[j], mask=les[j])
                    plsc.store_scatter(ibuf, [pos], gl + j * L, mask=les[j])
                return offs[G], gl + G * L
            cwm1, _ = lax.fori_loop(
                0, NB // G, sweep1, (zero_i - 1, ii))

            c_tau = cwm1[15] + 1
            ok1 = (c_tau >= K) & (c_tau <= CAPF)

            @pl.when(ok1)
            def _():
                ncb = (c_tau + (L - 1)) // L

                def ca(i, _):
                    merge(cbuf[pl.ds(i * L, L)])
                    return None
                lax.fori_loop(0, ncb, ca, None)

                tv = tref[...]
                cw2ref[...] = zero_i - 1

                def cb(i, _):
                    c = cbuf[pl.ds(i * L, L)]
                    le = c <= tv
                    lei = jnp.where(le, one_i, zero_i)
                    cums = lax.cumsum(lei)
                    cwm1c = cw2ref[...]
                    pos = cwm1c + cums
                    sel = le & (pos < K)
                    iv = ibuf[pl.ds(i * L, L)]
                    plsc.store_scatter(idxl, [pos], iv, mask=sel)
                    plsc.store_scatter(idxg[b], [pos], iv + row * N, mask=sel)
                    cw2ref[...] = cwm1c + jnp.full((L,), cums[15], jnp.int32)
                    return None
                lax.fori_loop(0, ncb, cb, None)
                flag[0] = (cw2ref[...][15] + 1 == K).astype(jnp.int32)

            @pl.when(flag[0] == 0)
            def _():
                for j in range(KB):
                    best[pl.ds(j * L, L)] = inf
                tref[...] = inf

                def pass_a(i, _):
                    tv = tref[...]
                    d = db[pl.ds(i * L, L)]
                    cnt = plsc.all_reduce_population_count(d < tv)

                    @pl.when(cnt[0] > 0)
                    def _():
                        merge(d)

                    return None
                lax.fori_loop(0, NB, pass_a, None)

                tv = tref[...]

                def b1(i, acc):
                    d = db[pl.ds(i * L, L)]
                    return acc + (d < tv).astype(jnp.int32)
                c_less = jnp.sum(
                    lax.fori_loop(0, NB, b1, zero_i))
                m = K - c_less

                def b2(i, carry):
                    cw, ct = carry
                    d = db[pl.ds(i * L, L)]
                    lt = d < tv
                    eq = d == tv
                    eqi = eq.astype(jnp.int32)
                    ranks = ct + lax.cumsum(eqi) - eqi
                    sel = lt | (eq & (ranks < m))
                    seli = sel.astype(jnp.int32)
                    pos = cw + lax.cumsum(seli) - seli
                    gl = i * L + ii
                    plsc.store_scatter(idxl, [pos], gl, mask=sel)
                    plsc.store_scatter(idxg[b], [pos], gl + row * N, mask=sel)
                    return cw + jnp.sum(seli), ct + jnp.sum(eqi)
                lax.fori_loop(0, NB, b2, (jnp.int32(0), jnp.int32(0)))

            def packw(i, _):
                bs = mrow[b][pl.ds(i * 64, 64)]
                mwords[pl.ds(i * L, L)] = plsc.bitcast(bs, jnp.int32)
                return None
            lax.fori_loop(0, NW4 // L, packw, None)
            for j in range(KB):
                iv = idxl[pl.ds(j * L, L)]
                w = plsc.load_gather(mwords, [jnp.right_shift(iv, 2)])
                sh = jnp.left_shift(iv & 3, 3)
                mb[b][pl.ds(j * L, L)] = jnp.right_shift(w, sh) & 1

        def fire_gather(b):
            cp_g[b] = pltpu.async_copy(feats_hbm.at[idxg[b]], rows[b], semg[b])

        def fire_out(r, b):
            row = wid * RPW + r
            c1 = pltpu.async_copy(rows[b], outf_hbm.at[row], semo[b])
            c2 = pltpu.async_copy(mb[b], outm_hbm.at[row], semo[b])
            cp_o[b] = (c1, c2)

        def wait_in(b):
            pltpu.make_async_copy(dists_hbm.at[0], drow[b], semd[b]).wait()
            pltpu.make_async_copy(
                mask_hbm.at[pl.ds(0, N)], mrow[b], semd[b]).wait()

        def wait_out(b):
            pltpu.make_async_copy(rows[b], outf_hbm.at[0], semo[b]).wait()
            pltpu.make_async_copy(mb[b], outm_hbm.at[0], semo[b]).wait()

        def wait_gather(b):
            pltpu.make_async_copy(
                feats_hbm.at[idxg[b]], rows[b], semg[b]).wait()

        NIT = RPW // 2
        fire_in(0, 0)

        def row_pair(it, _):
            r0 = it * 2

            @pl.when(it > 0)
            def _():
                wait_gather(1)
                fire_out(r0 - 1, 1)

            fire_in(r0 + 1, 1)
            wait_in(0)

            @pl.when(it > 0)
            def _():
                wait_out(0)

            compute(r0, 0)
            fire_gather(0)

            @pl.when(it < NIT - 1)
            def _():
                fire_in(r0 + 2, 0)

            wait_in(1)

            @pl.when(it > 0)
            def _():
                wait_out(1)

            compute(r0 + 1, 1)
            fire_gather(1)
            wait_gather(0)
            fire_out(r0, 0)
            return None
        lax.fori_loop(0, NIT, row_pair, None)
        wait_gather(1)
        fire_out(RPW - 1, 1)
        wait_out(0)
        wait_out(1)

    return sc_fn(dists, featsflat, maskw)


def kernel(dists, feats, coord_mask):
    B, N = dists.shape
    D = feats.shape[2]
    K = min(48, N)
    featsflat = feats.reshape(B * N, D)
    maskr = coord_mask.astype(jnp.int8).reshape(B * N)
    outf, outm = _sc_topk_gather(dists, featsflat, maskr, B, N, D, K)
    return outf, outm != 0
